# baseline (speedup 1.0000x reference)
import functools

import jax
import jax.numpy as jnp
from jax import lax
from jax.experimental import pallas as pl
from jax.experimental.pallas import tpu as pltpu
from jax.experimental.pallas import tpu_sc as plsc

N_ROWS = 1000000
N_COLS = 100000
RANK_K = 32
BATCH = 16384

NC = 2
NS = 16
NW = NC * NS
BPW = BATCH // NW
CH = 128
NCH = BPW // CH
LANES = 16
NG = BPW // LANES


def _lfm_body(i_hbm, j_hbm, u_hbm, v_hbm, out_hbm,
              idx_i, idx_j, u_rows, v_rows, out_loc, sem):
  wid = lax.axis_index("s") * NC + lax.axis_index("c")
  base = wid * BPW

  for c in range(NCH):
    pltpu.sync_copy(i_hbm.at[pl.ds(base + c * CH, CH)], idx_i.at[c])
    pltpu.sync_copy(j_hbm.at[pl.ds(base + c * CH, CH)], idx_j.at[c])

  copies = []
  for c in range(NCH):
    copies.append(pltpu.async_copy(
        u_hbm.at[idx_i.at[c]], u_rows.at[pl.ds(c * CH, CH), :], sem))
    copies.append(pltpu.async_copy(
        v_hbm.at[idx_j.at[c]], v_rows.at[pl.ds(c * CH, CH), :], sem))
  for cp in copies:
    cp.wait()

  lane = lax.iota(jnp.int32, LANES)

  def group(g, carry):
    acc = jnp.zeros((LANES,), jnp.float32)
    for r in range(LANES):
      row = g * LANES + r
      p = (u_rows[row, pl.ds(0, LANES)] * v_rows[row, pl.ds(0, LANES)]
           + u_rows[row, pl.ds(LANES, LANES)] * v_rows[row, pl.ds(LANES, LANES)])
      s = jnp.sum(p)
      acc = jnp.where(lane == r, s, acc)
    out_loc[pl.ds(g * LANES, LANES)] = acc
    return carry

  lax.fori_loop(0, NG, group, 0)
  pltpu.sync_copy(out_loc, out_hbm.at[pl.ds(base, BPW)])


@functools.partial(
    pl.kernel,
    out_type=jax.ShapeDtypeStruct((BATCH,), jnp.float32),
    mesh=plsc.VectorSubcoreMesh(core_axis_name="c", subcore_axis_name="s"),
    compiler_params=pltpu.CompilerParams(
        needs_layout_passes=False, use_tc_tiling_on_sc=False),
    scratch_types=[
        pltpu.VMEM((NCH, CH), jnp.int32),
        pltpu.VMEM((NCH, CH), jnp.int32),
        pltpu.VMEM((BPW, RANK_K), jnp.float32),
        pltpu.VMEM((BPW, RANK_K), jnp.float32),
        pltpu.VMEM((BPW,), jnp.float32),
        pltpu.SemaphoreType.DMA,
    ],
)
def _lfm_kernel(*refs):
  _lfm_body(*refs)


def kernel(i, j, U, V):
  return _lfm_kernel(i, j, U, V)


# speedup vs baseline: 1.0023x; 1.0023x over previous
import functools

import jax
import jax.numpy as jnp
from jax import lax
from jax.experimental import pallas as pl
from jax.experimental.pallas import tpu as pltpu
from jax.experimental.pallas import tpu_sc as plsc

N_ROWS = 1000000
N_COLS = 100000
RANK_K = 32
BATCH = 16384

NC = 2
NS = 16
NW = NC * NS
BPW = BATCH // NW
CH = 32
NCH = BPW // CH
LANES = 16
NG = BPW // LANES


def _lfm_body(i_hbm, j_hbm, u_hbm, v_hbm, out_hbm,
              idx_i, idx_j, u_rows, v_rows, out_loc, sem):
  wid = lax.axis_index("s") * NC + lax.axis_index("c")
  base = wid * BPW

  with jax.named_scope("idx_stage"):
    pltpu.sync_copy(i_hbm.at[pl.ds(base, BPW)], idx_i)
    pltpu.sync_copy(j_hbm.at[pl.ds(base, BPW)], idx_j)

  with jax.named_scope("gather"):
    copies = []
    for c in range(NCH):
      sl = pl.ds(c * CH, CH)
      copies.append(pltpu.async_copy(
          u_hbm.at[idx_i.at[sl]], u_rows.at[sl, :], sem))
      copies.append(pltpu.async_copy(
          v_hbm.at[idx_j.at[sl]], v_rows.at[sl, :], sem))
    for cp in copies:
      cp.wait()

  with jax.named_scope("dot"):
    lane = lax.iota(jnp.int32, LANES)

    def group(g, carry):
      acc = jnp.zeros((LANES,), jnp.float32)
      for r in range(LANES):
        row = g * LANES + r
        p = (u_rows[row, pl.ds(0, LANES)] * v_rows[row, pl.ds(0, LANES)]
             + u_rows[row, pl.ds(LANES, LANES)]
             * v_rows[row, pl.ds(LANES, LANES)])
        s = jnp.sum(p)
        acc = jnp.where(lane == r, s, acc)
      out_loc[pl.ds(g * LANES, LANES)] = acc
      return carry

    lax.fori_loop(0, NG, group, 0)

  with jax.named_scope("out_copy"):
    pltpu.sync_copy(out_loc, out_hbm.at[pl.ds(base, BPW)])


@functools.partial(
    pl.kernel,
    out_type=jax.ShapeDtypeStruct((BATCH,), jnp.float32),
    mesh=plsc.VectorSubcoreMesh(core_axis_name="c", subcore_axis_name="s"),
    compiler_params=pltpu.CompilerParams(
        needs_layout_passes=False, use_tc_tiling_on_sc=False),
    scratch_types=[
        pltpu.VMEM((BPW,), jnp.int32),
        pltpu.VMEM((BPW,), jnp.int32),
        pltpu.VMEM((BPW, RANK_K), jnp.float32),
        pltpu.VMEM((BPW, RANK_K), jnp.float32),
        pltpu.VMEM((BPW,), jnp.float32),
        pltpu.SemaphoreType.DMA,
    ],
)
def _lfm_kernel(*refs):
  _lfm_body(*refs)


def kernel(i, j, U, V):
  return _lfm_kernel(i, j, U, V)
